# Initial kernel scaffold; baseline (speedup 1.0000x reference)
import jax
import jax.numpy as jnp
from jax.experimental import pallas as pl


def kernel(x, in_centers, in_widths, out_centers, out_scaling, out_bias, input_rules, output_rules):
    raise NotImplementedError("write your pallas kernel here")


# speedup vs baseline: 2.4872x; 2.4872x over previous
import functools

import jax
import jax.numpy as jnp
from jax import lax
from jax.experimental import pallas as pl
from jax.experimental.pallas import tpu as pltpu
from jax.experimental.pallas import tpu_sc as plsc

B, V, K, R, A = 1024, 8, 16, 8192, 6
C = V * K
L = 16
NTILES = 32
BPT = B // NTILES
NCHUNK = R // L


def _sc_rule_kernel(x_rep, c_flat, w_flat, idx_t, or_t, oc_pad):
    mesh = plsc.VectorSubcoreMesh(core_axis_name="c", subcore_axis_name="s")

    @functools.partial(
        pl.kernel,
        mesh=mesh,
        compiler_params=pltpu.CompilerParams(needs_layout_passes=False),
        out_type=jax.ShapeDtypeStruct((B * 3 * L,), jnp.float32),
        scratch_types=[
            pltpu.VMEM((BPT * C,), jnp.float32),
            pltpu.VMEM((BPT * C,), jnp.float32),
            pltpu.VMEM((C,), jnp.float32),
            pltpu.VMEM((C,), jnp.float32),
            pltpu.VMEM((A, R), jnp.int32),
            pltpu.VMEM((2, R), jnp.int32),
            pltpu.VMEM((C,), jnp.float32),
            pltpu.VMEM((BPT * 3 * L,), jnp.float32),
        ],
    )
    def k(x_hbm, c_hbm, w_hbm, idx_hbm, or_hbm, oc_hbm, out_hbm,
          xs, fz, cv, nv, idxv, orv, ocv, acc):
        wid = lax.axis_index("s") * 2 + lax.axis_index("c")

        pltpu.sync_copy(x_hbm.at[pl.ds(wid * BPT * C, BPT * C)], xs)
        pltpu.sync_copy(c_hbm, cv)
        pltpu.sync_copy(w_hbm, nv)
        pltpu.sync_copy(idx_hbm, idxv)
        pltpu.sync_copy(or_hbm, orv)
        pltpu.sync_copy(oc_hbm, ocv)

        for t in range(C // L):
            wv = nv[pl.ds(t * L, L)]
            nv[pl.ds(t * L, L)] = -0.5 / (wv * wv)

        zero = jnp.zeros((L,), jnp.float32)

        def fuzz_body(b, carry):
            boff = b * C
            for t in range(C // L):
                xv = xs[pl.ds(boff + t * L, L)]
                d = xv - cv[pl.ds(t * L, L)]
                fz[pl.ds(boff + t * L, L)] = jnp.exp(d * d * nv[pl.ds(t * L, L)])
            for j in range(3):
                acc[pl.ds(b * 3 * L + j * L, L)] = zero
            return carry

        lax.fori_loop(0, BPT, fuzz_body, 0)

        def chunk_body(i, carry):
            base = i * L
            ia = [idxv[a, pl.ds(base, L)] for a in range(A)]
            ow0 = plsc.load_gather(ocv, [orv[0, pl.ds(base, L)]])
            ow1 = plsc.load_gather(ocv, [orv[1, pl.ds(base, L)]])
            for b in range(BPT):
                boff = b * C
                w = plsc.load_gather(fz, [ia[0] + boff])
                for a in range(1, A):
                    w = jnp.minimum(w, plsc.load_gather(fz, [ia[a] + boff]))
                plsc.addupdate(acc.at[pl.ds(b * 3 * L, L)], w * ow0)
                plsc.addupdate(acc.at[pl.ds(b * 3 * L + L, L)], w * ow1)
                plsc.addupdate(acc.at[pl.ds(b * 3 * L + 2 * L, L)], w)
            return carry

        lax.fori_loop(0, NCHUNK, chunk_body, 0)

        pltpu.sync_copy(acc, out_hbm.at[pl.ds(wid * BPT * 3 * L, BPT * 3 * L)])

    return k(x_rep, c_flat, w_flat, idx_t, or_t, oc_pad)


def _tc_epilogue(acc_flat, out_scaling, out_bias):
    def body(a_ref, s_ref, b_ref, o_ref):
        a = a_ref[...]
        s0 = jnp.sum(a[:, 0:L], axis=1, keepdims=True)
        s1 = jnp.sum(a[:, L:2 * L], axis=1, keepdims=True)
        sd = jnp.sum(a[:, 2 * L:3 * L], axis=1, keepdims=True)
        denom = jnp.maximum(sd, 1e-12)
        z = jnp.concatenate([s0, s1], axis=1) / denom
        o_ref[...] = jnp.tanh(z) * s_ref[...] + b_ref[...]

    return pl.pallas_call(
        body,
        out_shape=jax.ShapeDtypeStruct((B, 2), jnp.float32),
    )(acc_flat, out_scaling, out_bias)


def kernel(x, in_centers, in_widths, out_centers, out_scaling, out_bias,
           input_rules, output_rules):
    x_rep = jnp.repeat(x, K, axis=1).reshape(B * C)
    c_flat = in_centers.reshape(C)
    w_flat = in_widths.reshape(C)
    idx_t = input_rules.T.reshape(A, R)
    or_t = output_rules.T.reshape(2, R)
    oc_pad = jnp.pad(out_centers, (0, C - out_centers.shape[0]))

    acc = _sc_rule_kernel(x_rep, c_flat, w_flat, idx_t, or_t, oc_pad)
    return _tc_epilogue(acc.reshape(B, 3 * L), out_scaling, out_bias)


# speedup vs baseline: 3.5962x; 1.4459x over previous
import functools

import jax
import jax.numpy as jnp
from jax import lax
from jax.experimental import pallas as pl
from jax.experimental.pallas import tpu as pltpu
from jax.experimental.pallas import tpu_sc as plsc

B, V, K, R, A = 1024, 8, 16, 8192, 6
C = V * K
L = 16
NTILES = 32
BPT = B // NTILES
NCHUNK = R // L


def _sc_rule_kernel(x_rep, c_flat, w_flat, idx_t, or_t, oc_pad):
    mesh = plsc.VectorSubcoreMesh(core_axis_name="c", subcore_axis_name="s")

    @functools.partial(
        pl.kernel,
        mesh=mesh,
        compiler_params=pltpu.CompilerParams(needs_layout_passes=False),
        out_type=jax.ShapeDtypeStruct((B * 3 * L,), jnp.float32),
        scratch_types=[
            pltpu.VMEM((BPT * C,), jnp.float32),
            pltpu.VMEM((BPT * C,), jnp.float32),
            pltpu.VMEM((C,), jnp.float32),
            pltpu.VMEM((C,), jnp.float32),
            pltpu.VMEM((A, R), jnp.int32),
            pltpu.VMEM((2, R), jnp.int32),
            pltpu.VMEM((C,), jnp.float32),
            pltpu.VMEM((BPT * 3 * L,), jnp.float32),
        ],
    )
    def k(x_hbm, c_hbm, w_hbm, idx_hbm, or_hbm, oc_hbm, out_hbm,
          xs, fz, cv, nv, idxv, orv, ocv, acc):
        wid = lax.axis_index("s") * 2 + lax.axis_index("c")

        pltpu.sync_copy(x_hbm.at[pl.ds(wid * BPT * C, BPT * C)], xs)
        pltpu.sync_copy(c_hbm, cv)
        pltpu.sync_copy(w_hbm, nv)
        pltpu.sync_copy(idx_hbm, idxv)
        pltpu.sync_copy(or_hbm, orv)
        pltpu.sync_copy(oc_hbm, ocv)

        for t in range(C // L):
            wv = nv[pl.ds(t * L, L)]
            nv[pl.ds(t * L, L)] = -0.5 / (wv * wv)

        zero = jnp.zeros((L,), jnp.float32)

        def fuzz_body(b, carry):
            boff = b * C
            for t in range(C // L):
                xv = xs[pl.ds(boff + t * L, L)]
                d = xv - cv[pl.ds(t * L, L)]
                fz[pl.ds(boff + t * L, L)] = jnp.exp(d * d * nv[pl.ds(t * L, L)])
            for j in range(3):
                acc[pl.ds(b * 3 * L + j * L, L)] = zero
            return carry

        lax.fori_loop(0, BPT, fuzz_body, 0)

        @plsc.parallel_loop(0, NCHUNK)
        def chunk_body(i):
            base = i * L
            ia = [idxv[a, pl.ds(base, L)] for a in range(A)]
            ow0 = plsc.load_gather(ocv, [orv[0, pl.ds(base, L)]])
            ow1 = plsc.load_gather(ocv, [orv[1, pl.ds(base, L)]])
            for b in range(BPT):
                boff = b * C
                g = [plsc.load_gather(fz, [ia[a] + boff]) for a in range(A)]
                w01 = jnp.minimum(g[0], g[1])
                w23 = jnp.minimum(g[2], g[3])
                w45 = jnp.minimum(g[4], g[5])
                w = jnp.minimum(jnp.minimum(w01, w23), w45)
                plsc.addupdate(acc.at[pl.ds(b * 3 * L, L)], w * ow0)
                plsc.addupdate(acc.at[pl.ds(b * 3 * L + L, L)], w * ow1)
                plsc.addupdate(acc.at[pl.ds(b * 3 * L + 2 * L, L)], w)

        pltpu.sync_copy(acc, out_hbm.at[pl.ds(wid * BPT * 3 * L, BPT * 3 * L)])

    return k(x_rep, c_flat, w_flat, idx_t, or_t, oc_pad)


def _tc_epilogue(acc_flat, out_scaling, out_bias):
    def body(a_ref, s_ref, b_ref, o_ref):
        a = a_ref[...]
        s0 = jnp.sum(a[:, 0:L], axis=1, keepdims=True)
        s1 = jnp.sum(a[:, L:2 * L], axis=1, keepdims=True)
        sd = jnp.sum(a[:, 2 * L:3 * L], axis=1, keepdims=True)
        denom = jnp.maximum(sd, 1e-12)
        z = jnp.concatenate([s0, s1], axis=1) / denom
        o_ref[...] = jnp.tanh(z) * s_ref[...] + b_ref[...]

    return pl.pallas_call(
        body,
        out_shape=jax.ShapeDtypeStruct((B, 2), jnp.float32),
    )(acc_flat, out_scaling, out_bias)


def kernel(x, in_centers, in_widths, out_centers, out_scaling, out_bias,
           input_rules, output_rules):
    x_rep = jnp.repeat(x, K, axis=1).reshape(B * C)
    c_flat = in_centers.reshape(C)
    w_flat = in_widths.reshape(C)
    idx_t = input_rules.T.reshape(A, R)
    or_t = output_rules.T.reshape(2, R)
    oc_pad = jnp.pad(out_centers, (0, C - out_centers.shape[0]))

    acc = _sc_rule_kernel(x_rep, c_flat, w_flat, idx_t, or_t, oc_pad)
    return _tc_epilogue(acc.reshape(B, 3 * L), out_scaling, out_bias)


# speedup vs baseline: 4.5044x; 1.2525x over previous
import functools

import jax
import jax.numpy as jnp
from jax import lax
from jax.experimental import pallas as pl
from jax.experimental.pallas import tpu as pltpu
from jax.experimental.pallas import tpu_sc as plsc

B, V, K, R, A = 1024, 8, 16, 8192, 6
C = V * K
L = 16
NTILES = 32
BPT = B // NTILES
NCHUNK = R // L


def _sc_rule_kernel(x_rep, c_flat, w_flat, idx_t, or_t, oc_pad):
    mesh = plsc.VectorSubcoreMesh(core_axis_name="c", subcore_axis_name="s")

    @functools.partial(
        pl.kernel,
        mesh=mesh,
        compiler_params=pltpu.CompilerParams(needs_layout_passes=False),
        out_type=jax.ShapeDtypeStruct((B * 3 * L,), jnp.float32),
        scratch_types=[
            pltpu.VMEM((BPT * C,), jnp.float32),
            pltpu.VMEM((BPT // 2 * C,), jnp.int32),
            pltpu.VMEM((C,), jnp.float32),
            pltpu.VMEM((C,), jnp.float32),
            pltpu.VMEM((A, R), jnp.int32),
            pltpu.VMEM((2, R), jnp.int32),
            pltpu.VMEM((C,), jnp.float32),
            pltpu.VMEM((BPT * 3 * L,), jnp.float32),
        ],
    )
    def k(x_hbm, c_hbm, w_hbm, idx_hbm, or_hbm, oc_hbm, out_hbm,
          xs, fz, cv, nv, idxv, orv, ocv, acc):
        wid = lax.axis_index("s") * 2 + lax.axis_index("c")

        pltpu.sync_copy(x_hbm.at[pl.ds(wid * BPT * C, BPT * C)], xs)
        pltpu.sync_copy(c_hbm, cv)
        pltpu.sync_copy(w_hbm, nv)
        pltpu.sync_copy(idx_hbm, idxv)
        pltpu.sync_copy(or_hbm, orv)
        pltpu.sync_copy(oc_hbm, ocv)

        for t in range(C // L):
            wv = nv[pl.ds(t * L, L)]
            nv[pl.ds(t * L, L)] = -0.5 / (wv * wv)

        zero = jnp.zeros((L,), jnp.float32)

        def fuzz_body(bp, carry):
            for t in range(C // L):
                col = pl.ds(t * L, L)
                cvt = cv[col]
                nvt = nv[col]
                xe = xs[pl.ds((2 * bp) * C + t * L, L)]
                xo = xs[pl.ds((2 * bp + 1) * C + t * L, L)]
                de = xe - cvt
                do = xo - cvt
                fe = jnp.exp(de * de * nvt)
                fo = jnp.exp(do * do * nvt)
                packed = plsc.pack(fe, fo, format=plsc.PackFormat.INTERLEAVED)
                fz[pl.ds(bp * C + t * L, L)] = plsc.bitcast(packed, jnp.int32)
            for j in range(6):
                acc[pl.ds(bp * 6 * L + j * L, L)] = zero
            return carry

        lax.fori_loop(0, BPT // 2, fuzz_body, 0)

        @plsc.parallel_loop(0, NCHUNK)
        def chunk_body(i):
            base = i * L
            ia = [idxv[a, pl.ds(base, L)] for a in range(A)]
            ow0 = plsc.load_gather(ocv, [orv[0, pl.ds(base, L)]])
            ow1 = plsc.load_gather(ocv, [orv[1, pl.ds(base, L)]])
            for bp in range(BPT // 2):
                boff = bp * C
                g = [
                    plsc.bitcast(
                        plsc.load_gather(fz, [ia[a] + boff]), jnp.bfloat16
                    )
                    for a in range(A)
                ]
                w01 = jnp.minimum(g[0], g[1])
                w23 = jnp.minimum(g[2], g[3])
                w45 = jnp.minimum(g[4], g[5])
                w = jnp.minimum(jnp.minimum(w01, w23), w45)
                we, wo = plsc.unpack(w, format=plsc.PackFormat.INTERLEAVED)
                o = bp * 6 * L
                plsc.addupdate(acc.at[pl.ds(o, L)], we * ow0)
                plsc.addupdate(acc.at[pl.ds(o + L, L)], we * ow1)
                plsc.addupdate(acc.at[pl.ds(o + 2 * L, L)], we)
                plsc.addupdate(acc.at[pl.ds(o + 3 * L, L)], wo * ow0)
                plsc.addupdate(acc.at[pl.ds(o + 4 * L, L)], wo * ow1)
                plsc.addupdate(acc.at[pl.ds(o + 5 * L, L)], wo)

        pltpu.sync_copy(acc, out_hbm.at[pl.ds(wid * BPT * 3 * L, BPT * 3 * L)])

    return k(x_rep, c_flat, w_flat, idx_t, or_t, oc_pad)


def _tc_epilogue(acc_flat, out_scaling, out_bias):
    def body(a_ref, s_ref, b_ref, o_ref):
        a = a_ref[...]
        s0 = jnp.sum(a[:, 0:L], axis=1, keepdims=True)
        s1 = jnp.sum(a[:, L:2 * L], axis=1, keepdims=True)
        sd = jnp.sum(a[:, 2 * L:3 * L], axis=1, keepdims=True)
        denom = jnp.maximum(sd, 1e-12)
        z = jnp.concatenate([s0, s1], axis=1) / denom
        o_ref[...] = jnp.tanh(z) * s_ref[...] + b_ref[...]

    return pl.pallas_call(
        body,
        out_shape=jax.ShapeDtypeStruct((B, 2), jnp.float32),
    )(acc_flat, out_scaling, out_bias)


def kernel(x, in_centers, in_widths, out_centers, out_scaling, out_bias,
           input_rules, output_rules):
    x_rep = jnp.repeat(x, K, axis=1).reshape(B * C)
    c_flat = in_centers.reshape(C)
    w_flat = in_widths.reshape(C)
    idx_t = input_rules.T.reshape(A, R)
    or_t = output_rules.T.reshape(2, R)
    oc_pad = jnp.pad(out_centers, (0, C - out_centers.shape[0]))

    acc = _sc_rule_kernel(x_rep, c_flat, w_flat, idx_t, or_t, oc_pad)
    return _tc_epilogue(acc.reshape(B, 3 * L), out_scaling, out_bias)


# speedup vs baseline: 4.5577x; 1.0118x over previous
import functools

import jax
import jax.numpy as jnp
from jax import lax
from jax.experimental import pallas as pl
from jax.experimental.pallas import tpu as pltpu
from jax.experimental.pallas import tpu_sc as plsc

B, V, K, R, A = 1024, 8, 16, 8192, 6
C = V * K
L = 16
NTILES = 32
BPT = B // NTILES
NCHUNK = R // L


def _sc_rule_kernel(x_rep, c_flat, w_flat, idx_t, or_t, oc_pad):
    mesh = plsc.VectorSubcoreMesh(core_axis_name="c", subcore_axis_name="s")

    @functools.partial(
        pl.kernel,
        mesh=mesh,
        compiler_params=pltpu.CompilerParams(needs_layout_passes=False),
        out_type=jax.ShapeDtypeStruct((B * 3 * L,), jnp.float32),
        scratch_types=[
            pltpu.VMEM((BPT * C,), jnp.float32),
            pltpu.VMEM((BPT // 2 * C,), jnp.int32),
            pltpu.VMEM((C,), jnp.float32),
            pltpu.VMEM((C,), jnp.float32),
            pltpu.VMEM((A, R), jnp.int32),
            pltpu.VMEM((2, R), jnp.int32),
            pltpu.VMEM((C,), jnp.float32),
            pltpu.VMEM((BPT * 3 * L,), jnp.float32),
        ],
    )
    def k(x_hbm, c_hbm, w_hbm, idx_hbm, or_hbm, oc_hbm, out_hbm,
          xs, fz, cv, nv, idxv, orv, ocv, acc):
        wid = lax.axis_index("s") * 2 + lax.axis_index("c")

        pltpu.sync_copy(x_hbm.at[pl.ds(wid * BPT * C, BPT * C)], xs)
        pltpu.sync_copy(c_hbm, cv)
        pltpu.sync_copy(w_hbm, nv)
        pltpu.sync_copy(idx_hbm, idxv)
        pltpu.sync_copy(or_hbm, orv)
        pltpu.sync_copy(oc_hbm, ocv)

        for t in range(C // L):
            wv = nv[pl.ds(t * L, L)]
            nv[pl.ds(t * L, L)] = -0.5 / (wv * wv)

        zero = jnp.zeros((L,), jnp.float32)

        def fuzz_body(bp, carry):
            for t in range(C // L):
                col = pl.ds(t * L, L)
                cvt = cv[col]
                nvt = nv[col]
                xe = xs[pl.ds((2 * bp) * C + t * L, L)]
                xo = xs[pl.ds((2 * bp + 1) * C + t * L, L)]
                de = xe - cvt
                do = xo - cvt
                fe = jnp.exp(de * de * nvt)
                fo = jnp.exp(do * do * nvt)
                packed = plsc.pack(fe, fo, format=plsc.PackFormat.INTERLEAVED)
                fz[pl.ds(bp * C + t * L, L)] = plsc.bitcast(packed, jnp.int32)
            for j in range(6):
                acc[pl.ds(bp * 6 * L + j * L, L)] = zero
            return carry

        lax.fori_loop(0, BPT // 2, fuzz_body, 0)

        @plsc.parallel_loop(0, NCHUNK, unroll=4)
        def chunk_body(i):
            base = i * L
            ia = [idxv[a, pl.ds(base, L)] for a in range(A)]
            ow0 = plsc.load_gather(ocv, [orv[0, pl.ds(base, L)]])
            ow1 = plsc.load_gather(ocv, [orv[1, pl.ds(base, L)]])
            for bp in range(BPT // 2):
                boff = bp * C
                g = [
                    plsc.bitcast(
                        plsc.load_gather(fz, [ia[a] + boff]), jnp.bfloat16
                    )
                    for a in range(A)
                ]
                w01 = jnp.minimum(g[0], g[1])
                w23 = jnp.minimum(g[2], g[3])
                w45 = jnp.minimum(g[4], g[5])
                w = jnp.minimum(jnp.minimum(w01, w23), w45)
                we, wo = plsc.unpack(w, format=plsc.PackFormat.INTERLEAVED)
                o = bp * 6 * L
                plsc.addupdate(acc.at[pl.ds(o, L)], we * ow0)
                plsc.addupdate(acc.at[pl.ds(o + L, L)], we * ow1)
                plsc.addupdate(acc.at[pl.ds(o + 2 * L, L)], we)
                plsc.addupdate(acc.at[pl.ds(o + 3 * L, L)], wo * ow0)
                plsc.addupdate(acc.at[pl.ds(o + 4 * L, L)], wo * ow1)
                plsc.addupdate(acc.at[pl.ds(o + 5 * L, L)], wo)

        pltpu.sync_copy(acc, out_hbm.at[pl.ds(wid * BPT * 3 * L, BPT * 3 * L)])

    return k(x_rep, c_flat, w_flat, idx_t, or_t, oc_pad)


def _tc_epilogue(acc_flat, out_scaling, out_bias):
    def body(a_ref, s_ref, b_ref, o_ref):
        a = a_ref[...]
        s0 = jnp.sum(a[:, 0:L], axis=1, keepdims=True)
        s1 = jnp.sum(a[:, L:2 * L], axis=1, keepdims=True)
        sd = jnp.sum(a[:, 2 * L:3 * L], axis=1, keepdims=True)
        denom = jnp.maximum(sd, 1e-12)
        z = jnp.concatenate([s0, s1], axis=1) / denom
        o_ref[...] = jnp.tanh(z) * s_ref[...] + b_ref[...]

    return pl.pallas_call(
        body,
        out_shape=jax.ShapeDtypeStruct((B, 2), jnp.float32),
    )(acc_flat, out_scaling, out_bias)


def kernel(x, in_centers, in_widths, out_centers, out_scaling, out_bias,
           input_rules, output_rules):
    x_rep = jnp.repeat(x, K, axis=1).reshape(B * C)
    c_flat = in_centers.reshape(C)
    w_flat = in_widths.reshape(C)
    idx_t = input_rules.T.reshape(A, R)
    or_t = output_rules.T.reshape(2, R)
    oc_pad = jnp.pad(out_centers, (0, C - out_centers.shape[0]))

    acc = _sc_rule_kernel(x_rep, c_flat, w_flat, idx_t, or_t, oc_pad)
    return _tc_epilogue(acc.reshape(B, 3 * L), out_scaling, out_bias)
